# Initial kernel scaffold; baseline (speedup 1.0000x reference)
import jax
import jax.numpy as jnp
from jax.experimental import pallas as pl


def kernel(input, tables):
    raise NotImplementedError("write your pallas kernel here")


# speedup vs baseline: 1.1435x; 1.1435x over previous
import functools

import jax
import jax.numpy as jnp
from jax import lax
from jax.experimental import pallas as pl
from jax.experimental.pallas import tpu as pltpu
from jax.experimental.pallas import tpu_sc as plsc

_C = 26
_V = 100000
_D = 32
_B = 16384
_N = _B * _C
_NW = 32
_N_PER_W = _N // _NW
_CHUNK = 1024
_N_CHUNKS = _N_PER_W // _CHUNK

_mesh = plsc.VectorSubcoreMesh(core_axis_name="c", subcore_axis_name="s")


@functools.partial(
    pl.kernel,
    mesh=_mesh,
    out_type=jax.ShapeDtypeStruct((_N, _D), jnp.float32),
    scratch_types=[
        pltpu.VMEM((_CHUNK,), jnp.int32),
        pltpu.VMEM((_CHUNK, _D), jnp.float32),
        pltpu.SemaphoreType.DMA,
    ],
    compiler_params=pltpu.CompilerParams(use_tc_tiling_on_sc=False),
)
def _gather_all(idx_hbm, tab_hbm, out_hbm, idx_v, rows_v, sem):
    wid = lax.axis_index("s") * 2 + lax.axis_index("c")
    base = wid * _N_PER_W

    def body(k, carry):
        off = pl.multiple_of(base + k * _CHUNK, _CHUNK)
        pltpu.sync_copy(idx_hbm.at[pl.ds(off, _CHUNK)], idx_v)
        pltpu.async_copy(tab_hbm.at[idx_v], rows_v, sem).wait()
        pltpu.sync_copy(rows_v, out_hbm.at[pl.ds(off, _CHUNK)])
        return carry

    lax.fori_loop(0, _N_CHUNKS, body, 0)


def kernel(input, tables):
    flat_idx = (input.astype(jnp.int32)
                + jnp.arange(_C, dtype=jnp.int32)[None, :] * _V).reshape(_N)
    flat_tab = tables.reshape(_C * _V, _D)
    out = _gather_all(flat_idx, flat_tab)
    return out.reshape(_B, _C, _D)


# speedup vs baseline: 1.1453x; 1.0015x over previous
import functools

import jax
import jax.numpy as jnp
from jax import lax
from jax.experimental import pallas as pl
from jax.experimental.pallas import tpu as pltpu
from jax.experimental.pallas import tpu_sc as plsc

_C = 26
_V = 100000
_D = 32
_B = 16384
_NW = 32
_BLK = 1024
_NB = _B // _BLK
_NTASK = _C * _NB
_TASKS_PER_W = _NTASK // _NW

_mesh = plsc.VectorSubcoreMesh(core_axis_name="c", subcore_axis_name="s")


@functools.partial(
    pl.kernel,
    mesh=_mesh,
    out_type=jax.ShapeDtypeStruct((_B, _C, _D), jnp.float32),
    scratch_types=[
        pltpu.VMEM((_BLK,), jnp.int32),
        pltpu.VMEM((_BLK, _D), jnp.float32),
        pltpu.SemaphoreType.DMA,
    ],
    compiler_params=pltpu.CompilerParams(use_tc_tiling_on_sc=False),
)
def _gather_all(idx_hbm, tab_hbm, out_hbm, idx_v, rows_v, sem):
    wid = lax.axis_index("s") * 2 + lax.axis_index("c")

    def body(t, carry):
        tid = wid + t * _NW
        cat = tid // _NB
        b0 = (tid % _NB) * _BLK
        pltpu.sync_copy(idx_hbm.at[cat, pl.ds(b0, _BLK)], idx_v)
        pltpu.async_copy(tab_hbm.at[cat].at[idx_v], rows_v, sem).wait()
        pltpu.sync_copy(rows_v, out_hbm.at[pl.ds(b0, _BLK), cat])
        return carry

    lax.fori_loop(0, _TASKS_PER_W, body, 0)


def kernel(input, tables):
    idx_t = input.astype(jnp.int32).T
    return _gather_all(idx_t, tables)
